# baseline (speedup 1.0000x reference)
import jax
import jax.numpy as jnp
from jax import lax
from jax.experimental import pallas as pl
from jax.experimental.pallas import tpu as pltpu
from jax.experimental.pallas import tpu_sc as plsc

N = 65536
E = 262144
NC, NS = 2, 16
NW = NC * NS
K = 128
EPT = E // NW
NCH = EPT // K
NPT = N // NW
NPS = N // NS

_f32 = jnp.float32
_i32 = jnp.int32


def _lrelu(a):
    return jnp.maximum(a, 0.2 * a)


def _make_edge_pass(heads):
    rw = 2 * heads

    def body(*refs):
        (v_hbm, src_hbm, dst_hbm, coef_hbm, out_hbm,
         v_v, src_v, dst_v, coef_v, stag_v, zbuf_v,
         sem, semb, semc, semd) = refs[:15]
        accs = refs[15:]
        cid = lax.axis_index("c")
        sid = lax.axis_index("s")
        wid = cid * NS + sid
        pltpu.sync_copy(src_hbm.at[pl.ds(wid * NCH, NCH)], src_v)
        pltpu.sync_copy(dst_hbm.at[pl.ds(wid * NCH, NCH)], dst_v)
        pltpu.sync_copy(v_hbm, v_v)
        pltpu.sync_copy(coef_hbm, coef_v)

        zero16 = jnp.zeros((16,), _f32)
        for i in range(NPS // 16):
            zbuf_v[pl.ds(i * 16, 16)] = zero16
        for h in range(rw):
            pltpu.async_copy(zbuf_v, accs[h].at[pl.ds(sid * NPS, NPS)], sem)
        for h in range(rw):
            pltpu.make_async_copy(
                zbuf_v, accs[h].at[pl.ds(sid * NPS, NPS)], sem).wait()
        plsc.subcore_barrier()

        cs = [coef_v[h] for h in range(heads)]
        cd = [coef_v[heads + h] for h in range(heads)]

        bsem = [sem, semb, semc, semd]

        def do_chunk(ch, buf):
            for j in range(K // 16):
                si = src_v[ch, pl.ds(j * 16, 16)]
                di = dst_v[ch, pl.ds(j * 16, 16)]
                vs = plsc.load_gather(v_v, [si])
                vd = plsc.load_gather(v_v, [di])
                for h in range(heads):
                    p = jnp.exp(_lrelu(vs * cs[h] + vd * cd[h]))
                    stag_v[buf, h, pl.ds(j * 16, 16)] = p
                    stag_v[buf, heads + h, pl.ds(j * 16, 16)] = p * vs
            idx = dst_v.at[ch]
            for h in range(rw):
                pltpu.async_copy(stag_v.at[buf, h], accs[h].at[idx],
                                 bsem[buf], add=True)

        def drain_chunk(ch, buf):
            idx = dst_v.at[ch]
            for h in range(rw):
                pltpu.make_async_copy(stag_v.at[buf, h], accs[h].at[idx],
                                      bsem[buf]).wait()

        def chunk4(i, carry):
            ch = i * 4
            for b in range(4):
                @pl.when(i > 0)
                def _(b=b):
                    drain_chunk(ch + b, b)
                do_chunk(ch + b, b)
            return carry

        lax.fori_loop(0, NCH // 4, chunk4, 0)
        for b in range(4):
            drain_chunk(b, b)
        plsc.subcore_barrier()
        for h in range(rw):
            pltpu.async_copy(accs[h].at[pl.ds(sid * NPS, NPS)],
                             out_hbm.at[cid, h, pl.ds(sid * NPS, NPS)], sem)
        for h in range(rw):
            pltpu.make_async_copy(
                accs[h].at[pl.ds(sid * NPS, NPS)],
                out_hbm.at[cid, h, pl.ds(sid * NPS, NPS)], sem).wait()

    mesh = plsc.VectorSubcoreMesh(core_axis_name="c", subcore_axis_name="s")
    return pl.kernel(
        body,
        out_type=jax.ShapeDtypeStruct((NC, rw, N), _f32),
        mesh=mesh,
        compiler_params=pltpu.CompilerParams(needs_layout_passes=False),
        scratch_types=[
            pltpu.VMEM((N,), _f32),
            pltpu.VMEM((NCH, K), _i32),
            pltpu.VMEM((NCH, K), _i32),
            pltpu.VMEM((2 * heads, 16), _f32),
            pltpu.VMEM((4, rw, K), _f32),
            pltpu.VMEM((NPS,), _f32),
            pltpu.SemaphoreType.DMA,
            pltpu.SemaphoreType.DMA,
            pltpu.SemaphoreType.DMA,
            pltpu.SemaphoreType.DMA,
        ] + [pltpu.VMEM_SHARED((N,), _f32) for _ in range(rw)],
    )


def _node1_body(acc_hbm, v_hbm, coef_hbm, out_hbm,
                accA_v, accB_v, v_v, coef_v, out_v, sem):
    cid = lax.axis_index("c")
    sid = lax.axis_index("s")
    wid = cid * NS + sid
    nb = wid * NPT
    for h in range(8):
        pltpu.async_copy(acc_hbm.at[0, h, pl.ds(nb, NPT)], accA_v.at[h], sem)
        pltpu.async_copy(acc_hbm.at[1, h, pl.ds(nb, NPT)], accB_v.at[h], sem)
    pltpu.async_copy(v_hbm.at[pl.ds(nb, NPT)], v_v, sem)
    pltpu.async_copy(coef_hbm, coef_v, sem)
    for h in range(8):
        pltpu.make_async_copy(acc_hbm.at[0, h, pl.ds(nb, NPT)],
                              accA_v.at[h], sem).wait()
        pltpu.make_async_copy(acc_hbm.at[1, h, pl.ds(nb, NPT)],
                              accB_v.at[h], sem).wait()
    pltpu.make_async_copy(v_hbm.at[pl.ds(nb, NPT)], v_v, sem).wait()
    pltpu.make_async_copy(coef_hbm, coef_v, sem).wait()
    csum = [coef_v[h] for h in range(4)]
    pco = [coef_v[4 + h] for h in range(4)]
    mco = [coef_v[8 + h] for h in range(4)]

    def grp(g, carry):
        xs = v_v[pl.ds(g * 16, 16)]
        acc = jnp.zeros((16,), _f32)
        for h in range(4):
            den = (accA_v[h, pl.ds(g * 16, 16)]
                   + accB_v[h, pl.ds(g * 16, 16)])
            num = (accA_v[4 + h, pl.ds(g * 16, 16)]
                   + accB_v[4 + h, pl.ds(g * 16, 16)])
            ps = jnp.exp(_lrelu(xs * csum[h]))
            s = (num + ps * xs) / (den + ps + 1e-16)
            acc = acc + jnp.where(s >= 0.0, s * pco[h], s * mco[h])
        out_v[pl.ds(g * 16, 16)] = acc
        return carry

    lax.fori_loop(0, NPT // 16, grp, 0)
    pltpu.sync_copy(out_v, out_hbm.at[pl.ds(nb, NPT)])


def _node2_body(acc_hbm, v_hbm, coef_hbm, out_hbm,
                accA_v, accB_v, v_v, coef_v, out_v, sem):
    cid = lax.axis_index("c")
    sid = lax.axis_index("s")
    wid = cid * NS + sid
    nb = wid * NPT
    for h in range(2):
        pltpu.async_copy(acc_hbm.at[0, h, pl.ds(nb, NPT)], accA_v.at[h], sem)
        pltpu.async_copy(acc_hbm.at[1, h, pl.ds(nb, NPT)], accB_v.at[h], sem)
    pltpu.async_copy(v_hbm.at[pl.ds(nb, NPT)], v_v, sem)
    pltpu.async_copy(coef_hbm, coef_v, sem)
    for h in range(2):
        pltpu.make_async_copy(acc_hbm.at[0, h, pl.ds(nb, NPT)],
                              accA_v.at[h], sem).wait()
        pltpu.make_async_copy(acc_hbm.at[1, h, pl.ds(nb, NPT)],
                              accB_v.at[h], sem).wait()
    pltpu.make_async_copy(v_hbm.at[pl.ds(nb, NPT)], v_v, sem).wait()
    pltpu.make_async_copy(coef_hbm, coef_v, sem).wait()
    csum2 = coef_v[0]
    bias2 = coef_v[1]

    def grp(g, carry):
        xs = v_v[pl.ds(g * 16, 16)]
        den = (accA_v[0, pl.ds(g * 16, 16)]
               + accB_v[0, pl.ds(g * 16, 16)])
        num = (accA_v[1, pl.ds(g * 16, 16)]
               + accB_v[1, pl.ds(g * 16, 16)])
        ps = jnp.exp(_lrelu(xs * csum2))
        s = (num + ps * xs) / (den + ps + 1e-16) + bias2
        out_v[pl.ds(g * 16, 16)] = s
        return carry

    lax.fori_loop(0, NPT // 16, grp, 0)
    pltpu.sync_copy(out_v, out_hbm.at[pl.ds(nb, NPT)])


def _make_node_pass(body, ncoef, rw):
    mesh = plsc.VectorSubcoreMesh(core_axis_name="c", subcore_axis_name="s")
    return pl.kernel(
        body,
        out_type=jax.ShapeDtypeStruct((N,), _f32),
        mesh=mesh,
        compiler_params=pltpu.CompilerParams(needs_layout_passes=False),
        scratch_types=[
            pltpu.VMEM((rw, NPT), _f32),
            pltpu.VMEM((rw, NPT), _f32),
            pltpu.VMEM((NPT,), _f32),
            pltpu.VMEM((ncoef, 16), _f32),
            pltpu.VMEM((NPT,), _f32),
            pltpu.SemaphoreType.DMA,
        ],
    )


def kernel(x, edge_index, W1, att_src1, att_dst1, bias1,
           W2, att_src2, att_dst2, bias2):
    xf = x.reshape(N).astype(_f32)
    src = edge_index[0].astype(_i32).reshape(NW * NCH, K)
    dst = edge_index[1].astype(_i32).reshape(NW * NCH, K)

    W1r = W1.reshape(4, 64)
    W2r = W2.reshape(4, 64)
    cs1 = jnp.sum(W1r * att_src1[0], axis=1)
    cd1 = jnp.sum(W1r * att_dst1[0], axis=1)
    prod = W1r * W2r
    pco = jnp.sum(jnp.where(W1r > 0, prod, 0.0), axis=1)
    mco = jnp.sum(jnp.where(W1r < 0, prod, 0.0), axis=1)
    cs2 = att_src2.reshape(())
    cd2 = att_dst2.reshape(())

    ones16 = jnp.ones((16,), _f32)
    coef_e1 = jnp.concatenate([cs1, cd1])[:, None] * ones16
    coef_n1 = jnp.concatenate([cs1 + cd1, pco, mco])[:, None] * ones16
    coef_e2 = jnp.stack([cs2, cd2])[:, None] * ones16
    coef_n2 = jnp.stack([cs2 + cd2, bias2.reshape(())])[:, None] * ones16

    edge1 = _make_edge_pass(4)
    edge2 = _make_edge_pass(1)
    node1 = _make_node_pass(_node1_body, 12, 8)
    node2 = _make_node_pass(_node2_body, 2, 2)

    acc1 = edge1(xf, src, dst, coef_e1)
    xp2 = node1(acc1, xf, coef_n1)
    acc2 = edge2(xp2, src, dst, coef_e2)
    out = node2(acc2, xp2, coef_n2)
    return out.reshape(1, 256, 256)


# speedup vs baseline: 1.0020x; 1.0020x over previous
import jax
import jax.numpy as jnp
from jax import lax
from jax.experimental import pallas as pl
from jax.experimental.pallas import tpu as pltpu
from jax.experimental.pallas import tpu_sc as plsc

N = 65536
E = 262144
NC, NS = 2, 16
NW = NC * NS
K = 128
EPT = E // NW
NCH = EPT // K
NPT = N // NW
NPS = N // NS

_f32 = jnp.float32
_i32 = jnp.int32


def _lrelu(a):
    return jnp.maximum(a, 0.2 * a)


def _make_edge_pass(heads):
    rw = 2 * heads

    def body(*refs):
        (v_hbm, src_hbm, dst_hbm, coef_hbm, out_hbm,
         v_v, src_v, dst_v, coef_v, stag_v, zbuf_v, sem, semb) = refs[:13]
        accs = refs[13:]
        cid = lax.axis_index("c")
        sid = lax.axis_index("s")
        wid = cid * NS + sid
        pltpu.sync_copy(src_hbm.at[pl.ds(wid * NCH, NCH)], src_v)
        pltpu.sync_copy(dst_hbm.at[pl.ds(wid * NCH, NCH)], dst_v)
        pltpu.sync_copy(v_hbm, v_v)
        pltpu.sync_copy(coef_hbm, coef_v)

        zero16 = jnp.zeros((16,), _f32)
        for i in range(NPS // 16):
            zbuf_v[pl.ds(i * 16, 16)] = zero16
        for h in range(rw):
            pltpu.async_copy(zbuf_v, accs[h].at[pl.ds(sid * NPS, NPS)], sem)
        for h in range(rw):
            pltpu.make_async_copy(
                zbuf_v, accs[h].at[pl.ds(sid * NPS, NPS)], sem).wait()
        plsc.subcore_barrier()

        cs = [coef_v[h] for h in range(heads)]
        cd = [coef_v[heads + h] for h in range(heads)]

        bsem = [sem, semb]

        def do_chunk(ch, buf):
            for j in range(K // 16):
                si = src_v[ch, pl.ds(j * 16, 16)]
                di = dst_v[ch, pl.ds(j * 16, 16)]
                vs = plsc.load_gather(v_v, [si])
                vd = plsc.load_gather(v_v, [di])
                for h in range(heads):
                    p = jnp.exp(_lrelu(vs * cs[h] + vd * cd[h]))
                    stag_v[buf, h, pl.ds(j * 16, 16)] = p
                    stag_v[buf, heads + h, pl.ds(j * 16, 16)] = p * vs
            idx = dst_v.at[ch]
            for h in range(rw):
                pltpu.async_copy(stag_v.at[buf, h], accs[h].at[idx],
                                 bsem[buf], add=True)

        def drain_chunk(ch, buf):
            idx = dst_v.at[ch]
            for h in range(rw):
                pltpu.make_async_copy(stag_v.at[buf, h], accs[h].at[idx],
                                      bsem[buf]).wait()

        def chunk2(i, carry):
            ch = i * 2

            @pl.when(i > 0)
            def _():
                drain_chunk(ch, 0)
            do_chunk(ch, 0)

            @pl.when(i > 0)
            def _():
                drain_chunk(ch + 1, 1)
            do_chunk(ch + 1, 1)
            return carry

        lax.fori_loop(0, NCH // 2, chunk2, 0)
        drain_chunk(0, 0)
        drain_chunk(1, 1)
        plsc.subcore_barrier()
        for h in range(rw):
            pltpu.async_copy(accs[h].at[pl.ds(sid * NPS, NPS)],
                             out_hbm.at[cid, h, pl.ds(sid * NPS, NPS)], sem)
        for h in range(rw):
            pltpu.make_async_copy(
                accs[h].at[pl.ds(sid * NPS, NPS)],
                out_hbm.at[cid, h, pl.ds(sid * NPS, NPS)], sem).wait()

    mesh = plsc.VectorSubcoreMesh(core_axis_name="c", subcore_axis_name="s")
    return pl.kernel(
        body,
        out_type=jax.ShapeDtypeStruct((NC, rw, N), _f32),
        mesh=mesh,
        compiler_params=pltpu.CompilerParams(needs_layout_passes=False),
        scratch_types=[
            pltpu.VMEM((N,), _f32),
            pltpu.VMEM((NCH, K), _i32),
            pltpu.VMEM((NCH, K), _i32),
            pltpu.VMEM((2 * heads, 16), _f32),
            pltpu.VMEM((2, rw, K), _f32),
            pltpu.VMEM((NPS,), _f32),
            pltpu.SemaphoreType.DMA,
            pltpu.SemaphoreType.DMA,
        ] + [pltpu.VMEM_SHARED((N,), _f32) for _ in range(rw)],
    )


def _node1_body(acc_hbm, v_hbm, coef_hbm, out_hbm,
                accA_v, accB_v, v_v, coef_v, out_v, sem):
    cid = lax.axis_index("c")
    sid = lax.axis_index("s")
    wid = cid * NS + sid
    nb = wid * NPT
    for h in range(8):
        pltpu.async_copy(acc_hbm.at[0, h, pl.ds(nb, NPT)], accA_v.at[h], sem)
        pltpu.async_copy(acc_hbm.at[1, h, pl.ds(nb, NPT)], accB_v.at[h], sem)
    pltpu.async_copy(v_hbm.at[pl.ds(nb, NPT)], v_v, sem)
    pltpu.async_copy(coef_hbm, coef_v, sem)
    for h in range(8):
        pltpu.make_async_copy(acc_hbm.at[0, h, pl.ds(nb, NPT)],
                              accA_v.at[h], sem).wait()
        pltpu.make_async_copy(acc_hbm.at[1, h, pl.ds(nb, NPT)],
                              accB_v.at[h], sem).wait()
    pltpu.make_async_copy(v_hbm.at[pl.ds(nb, NPT)], v_v, sem).wait()
    pltpu.make_async_copy(coef_hbm, coef_v, sem).wait()
    csum = [coef_v[h] for h in range(4)]
    pco = [coef_v[4 + h] for h in range(4)]
    mco = [coef_v[8 + h] for h in range(4)]

    def grp(g, carry):
        xs = v_v[pl.ds(g * 16, 16)]
        acc = jnp.zeros((16,), _f32)
        for h in range(4):
            den = (accA_v[h, pl.ds(g * 16, 16)]
                   + accB_v[h, pl.ds(g * 16, 16)])
            num = (accA_v[4 + h, pl.ds(g * 16, 16)]
                   + accB_v[4 + h, pl.ds(g * 16, 16)])
            ps = jnp.exp(_lrelu(xs * csum[h]))
            s = (num + ps * xs) / (den + ps + 1e-16)
            acc = acc + jnp.where(s >= 0.0, s * pco[h], s * mco[h])
        out_v[pl.ds(g * 16, 16)] = acc
        return carry

    lax.fori_loop(0, NPT // 16, grp, 0)
    pltpu.sync_copy(out_v, out_hbm.at[pl.ds(nb, NPT)])


def _node2_body(acc_hbm, v_hbm, coef_hbm, out_hbm,
                accA_v, accB_v, v_v, coef_v, out_v, sem):
    cid = lax.axis_index("c")
    sid = lax.axis_index("s")
    wid = cid * NS + sid
    nb = wid * NPT
    for h in range(2):
        pltpu.async_copy(acc_hbm.at[0, h, pl.ds(nb, NPT)], accA_v.at[h], sem)
        pltpu.async_copy(acc_hbm.at[1, h, pl.ds(nb, NPT)], accB_v.at[h], sem)
    pltpu.async_copy(v_hbm.at[pl.ds(nb, NPT)], v_v, sem)
    pltpu.async_copy(coef_hbm, coef_v, sem)
    for h in range(2):
        pltpu.make_async_copy(acc_hbm.at[0, h, pl.ds(nb, NPT)],
                              accA_v.at[h], sem).wait()
        pltpu.make_async_copy(acc_hbm.at[1, h, pl.ds(nb, NPT)],
                              accB_v.at[h], sem).wait()
    pltpu.make_async_copy(v_hbm.at[pl.ds(nb, NPT)], v_v, sem).wait()
    pltpu.make_async_copy(coef_hbm, coef_v, sem).wait()
    csum2 = coef_v[0]
    bias2 = coef_v[1]

    def grp(g, carry):
        xs = v_v[pl.ds(g * 16, 16)]
        den = (accA_v[0, pl.ds(g * 16, 16)]
               + accB_v[0, pl.ds(g * 16, 16)])
        num = (accA_v[1, pl.ds(g * 16, 16)]
               + accB_v[1, pl.ds(g * 16, 16)])
        ps = jnp.exp(_lrelu(xs * csum2))
        s = (num + ps * xs) / (den + ps + 1e-16) + bias2
        out_v[pl.ds(g * 16, 16)] = s
        return carry

    lax.fori_loop(0, NPT // 16, grp, 0)
    pltpu.sync_copy(out_v, out_hbm.at[pl.ds(nb, NPT)])


def _make_node_pass(body, ncoef, rw):
    mesh = plsc.VectorSubcoreMesh(core_axis_name="c", subcore_axis_name="s")
    return pl.kernel(
        body,
        out_type=jax.ShapeDtypeStruct((N,), _f32),
        mesh=mesh,
        compiler_params=pltpu.CompilerParams(needs_layout_passes=False),
        scratch_types=[
            pltpu.VMEM((rw, NPT), _f32),
            pltpu.VMEM((rw, NPT), _f32),
            pltpu.VMEM((NPT,), _f32),
            pltpu.VMEM((ncoef, 16), _f32),
            pltpu.VMEM((NPT,), _f32),
            pltpu.SemaphoreType.DMA,
        ],
    )


def kernel(x, edge_index, W1, att_src1, att_dst1, bias1,
           W2, att_src2, att_dst2, bias2):
    xf = x.reshape(N).astype(_f32)
    src = edge_index[0].astype(_i32).reshape(NW * NCH, K)
    dst = edge_index[1].astype(_i32).reshape(NW * NCH, K)

    W1r = W1.reshape(4, 64)
    W2r = W2.reshape(4, 64)
    cs1 = jnp.sum(W1r * att_src1[0], axis=1)
    cd1 = jnp.sum(W1r * att_dst1[0], axis=1)
    prod = W1r * W2r
    pco = jnp.sum(jnp.where(W1r > 0, prod, 0.0), axis=1)
    mco = jnp.sum(jnp.where(W1r < 0, prod, 0.0), axis=1)
    cs2 = att_src2.reshape(())
    cd2 = att_dst2.reshape(())

    ones16 = jnp.ones((16,), _f32)
    coef_e1 = jnp.concatenate([cs1, cd1])[:, None] * ones16
    coef_n1 = jnp.concatenate([cs1 + cd1, pco, mco])[:, None] * ones16
    coef_e2 = jnp.stack([cs2, cd2])[:, None] * ones16
    coef_n2 = jnp.stack([cs2 + cd2, bias2.reshape(())])[:, None] * ones16

    edge1 = _make_edge_pass(4)
    edge2 = _make_edge_pass(1)
    node1 = _make_node_pass(_node1_body, 12, 8)
    node2 = _make_node_pass(_node2_body, 2, 2)

    acc1 = edge1(xf, src, dst, coef_e1)
    xp2 = node1(acc1, xf, coef_n1)
    acc2 = edge2(xp2, src, dst, coef_e2)
    out = node2(acc2, xp2, coef_n2)
    return out.reshape(1, 256, 256)


# speedup vs baseline: 1.0335x; 1.0315x over previous
import jax
import jax.numpy as jnp
from jax import lax
from jax.experimental import pallas as pl
from jax.experimental.pallas import tpu as pltpu
from jax.experimental.pallas import tpu_sc as plsc

N = 65536
E = 262144
NC, NS = 2, 16
NW = NC * NS
K = 128
EPT = E // NW
NCH = EPT // K
NPT = N // NW
NPS = N // NS

_f32 = jnp.float32
_i32 = jnp.int32


def _lrelu(a):
    return jnp.maximum(a, 0.2 * a)


def _make_edge_pass(heads):
    rw = 2 * heads

    def body(*refs):
        (v_hbm, src_hbm, dst_hbm, coef_hbm, out_hbm,
         v_v, src_v, dst_v, coef_v, stag_v, zbuf_v, sem, semb) = refs[:13]
        accs = refs[13:]
        cid = lax.axis_index("c")
        sid = lax.axis_index("s")
        wid = cid * NS + sid
        pltpu.async_copy(src_hbm.at[pl.ds(wid * NCH, NCH)], src_v, sem)
        pltpu.async_copy(dst_hbm.at[pl.ds(wid * NCH, NCH)], dst_v, sem)
        pltpu.async_copy(v_hbm, v_v, sem)
        pltpu.async_copy(coef_hbm, coef_v, sem)
        pltpu.make_async_copy(src_hbm.at[pl.ds(wid * NCH, NCH)],
                              src_v, sem).wait()
        pltpu.make_async_copy(dst_hbm.at[pl.ds(wid * NCH, NCH)],
                              dst_v, sem).wait()
        pltpu.make_async_copy(v_hbm, v_v, sem).wait()
        pltpu.make_async_copy(coef_hbm, coef_v, sem).wait()

        zero16 = jnp.zeros((16,), _f32)
        for i in range(NPS // 16):
            zbuf_v[pl.ds(i * 16, 16)] = zero16
        for h in range(rw):
            pltpu.async_copy(zbuf_v, accs[h].at[pl.ds(sid * NPS, NPS)], sem)
        for h in range(rw):
            pltpu.make_async_copy(
                zbuf_v, accs[h].at[pl.ds(sid * NPS, NPS)], sem).wait()
        plsc.subcore_barrier()

        cs = [coef_v[h] for h in range(heads)]
        cd = [coef_v[heads + h] for h in range(heads)]

        bsem = [sem, semb]

        def do_chunk(ch, buf):
            for j in range(K // 16):
                si = src_v[ch, pl.ds(j * 16, 16)]
                di = dst_v[ch, pl.ds(j * 16, 16)]
                vs = plsc.load_gather(v_v, [si])
                vd = plsc.load_gather(v_v, [di])
                for h in range(heads):
                    p = jnp.exp(_lrelu(vs * cs[h] + vd * cd[h]))
                    stag_v[buf, h, pl.ds(j * 16, 16)] = p
                    stag_v[buf, heads + h, pl.ds(j * 16, 16)] = p * vs
            idx = dst_v.at[ch]
            for h in range(rw):
                pltpu.async_copy(stag_v.at[buf, h], accs[h].at[idx],
                                 bsem[buf], add=True)

        def drain_chunk(ch, buf):
            idx = dst_v.at[ch]
            for h in range(rw):
                pltpu.make_async_copy(stag_v.at[buf, h], accs[h].at[idx],
                                      bsem[buf]).wait()

        def chunk2(i, carry):
            ch = i * 2

            @pl.when(i > 0)
            def _():
                drain_chunk(ch, 0)
            do_chunk(ch, 0)

            @pl.when(i > 0)
            def _():
                drain_chunk(ch + 1, 1)
            do_chunk(ch + 1, 1)
            return carry

        lax.fori_loop(0, NCH // 2, chunk2, 0)
        drain_chunk(0, 0)
        drain_chunk(1, 1)
        plsc.subcore_barrier()
        for h in range(rw):
            pltpu.async_copy(accs[h].at[pl.ds(sid * NPS, NPS)],
                             out_hbm.at[cid, h, pl.ds(sid * NPS, NPS)], sem)
        for h in range(rw):
            pltpu.make_async_copy(
                accs[h].at[pl.ds(sid * NPS, NPS)],
                out_hbm.at[cid, h, pl.ds(sid * NPS, NPS)], sem).wait()

    mesh = plsc.VectorSubcoreMesh(core_axis_name="c", subcore_axis_name="s")
    return pl.kernel(
        body,
        out_type=jax.ShapeDtypeStruct((NC, rw, N), _f32),
        mesh=mesh,
        compiler_params=pltpu.CompilerParams(needs_layout_passes=False),
        scratch_types=[
            pltpu.VMEM((N,), _f32),
            pltpu.VMEM((NCH, K), _i32),
            pltpu.VMEM((NCH, K), _i32),
            pltpu.VMEM((2 * heads, 16), _f32),
            pltpu.VMEM((2, rw, K), _f32),
            pltpu.VMEM((NPS,), _f32),
            pltpu.SemaphoreType.DMA,
            pltpu.SemaphoreType.DMA,
        ] + [pltpu.VMEM_SHARED((N,), _f32) for _ in range(rw)],
    )


def _node1_body(acc_hbm, v_hbm, coef_hbm, out_hbm,
                accA_v, accB_v, v_v, coef_v, out_v, sem):
    cid = lax.axis_index("c")
    sid = lax.axis_index("s")
    wid = cid * NS + sid
    nb = wid * NPT
    for h in range(8):
        pltpu.async_copy(acc_hbm.at[0, h, pl.ds(nb, NPT)], accA_v.at[h], sem)
        pltpu.async_copy(acc_hbm.at[1, h, pl.ds(nb, NPT)], accB_v.at[h], sem)
    pltpu.async_copy(v_hbm.at[pl.ds(nb, NPT)], v_v, sem)
    pltpu.async_copy(coef_hbm, coef_v, sem)
    for h in range(8):
        pltpu.make_async_copy(acc_hbm.at[0, h, pl.ds(nb, NPT)],
                              accA_v.at[h], sem).wait()
        pltpu.make_async_copy(acc_hbm.at[1, h, pl.ds(nb, NPT)],
                              accB_v.at[h], sem).wait()
    pltpu.make_async_copy(v_hbm.at[pl.ds(nb, NPT)], v_v, sem).wait()
    pltpu.make_async_copy(coef_hbm, coef_v, sem).wait()
    csum = [coef_v[h] for h in range(4)]
    pco = [coef_v[4 + h] for h in range(4)]
    mco = [coef_v[8 + h] for h in range(4)]

    def grp(g, carry):
        xs = v_v[pl.ds(g * 16, 16)]
        acc = jnp.zeros((16,), _f32)
        for h in range(4):
            den = (accA_v[h, pl.ds(g * 16, 16)]
                   + accB_v[h, pl.ds(g * 16, 16)])
            num = (accA_v[4 + h, pl.ds(g * 16, 16)]
                   + accB_v[4 + h, pl.ds(g * 16, 16)])
            ps = jnp.exp(_lrelu(xs * csum[h]))
            s = (num + ps * xs) / (den + ps + 1e-16)
            acc = acc + jnp.where(s >= 0.0, s * pco[h], s * mco[h])
        out_v[pl.ds(g * 16, 16)] = acc
        return carry

    lax.fori_loop(0, NPT // 16, grp, 0)
    pltpu.sync_copy(out_v, out_hbm.at[pl.ds(nb, NPT)])


def _node2_body(acc_hbm, v_hbm, coef_hbm, out_hbm,
                accA_v, accB_v, v_v, coef_v, out_v, sem):
    cid = lax.axis_index("c")
    sid = lax.axis_index("s")
    wid = cid * NS + sid
    nb = wid * NPT
    for h in range(2):
        pltpu.async_copy(acc_hbm.at[0, h, pl.ds(nb, NPT)], accA_v.at[h], sem)
        pltpu.async_copy(acc_hbm.at[1, h, pl.ds(nb, NPT)], accB_v.at[h], sem)
    pltpu.async_copy(v_hbm.at[pl.ds(nb, NPT)], v_v, sem)
    pltpu.async_copy(coef_hbm, coef_v, sem)
    for h in range(2):
        pltpu.make_async_copy(acc_hbm.at[0, h, pl.ds(nb, NPT)],
                              accA_v.at[h], sem).wait()
        pltpu.make_async_copy(acc_hbm.at[1, h, pl.ds(nb, NPT)],
                              accB_v.at[h], sem).wait()
    pltpu.make_async_copy(v_hbm.at[pl.ds(nb, NPT)], v_v, sem).wait()
    pltpu.make_async_copy(coef_hbm, coef_v, sem).wait()
    csum2 = coef_v[0]
    bias2 = coef_v[1]

    def grp(g, carry):
        xs = v_v[pl.ds(g * 16, 16)]
        den = (accA_v[0, pl.ds(g * 16, 16)]
               + accB_v[0, pl.ds(g * 16, 16)])
        num = (accA_v[1, pl.ds(g * 16, 16)]
               + accB_v[1, pl.ds(g * 16, 16)])
        ps = jnp.exp(_lrelu(xs * csum2))
        s = (num + ps * xs) / (den + ps + 1e-16) + bias2
        out_v[pl.ds(g * 16, 16)] = s
        return carry

    lax.fori_loop(0, NPT // 16, grp, 0)
    pltpu.sync_copy(out_v, out_hbm.at[pl.ds(nb, NPT)])


def _make_node_pass(body, ncoef, rw):
    mesh = plsc.VectorSubcoreMesh(core_axis_name="c", subcore_axis_name="s")
    return pl.kernel(
        body,
        out_type=jax.ShapeDtypeStruct((N,), _f32),
        mesh=mesh,
        compiler_params=pltpu.CompilerParams(needs_layout_passes=False),
        scratch_types=[
            pltpu.VMEM((rw, NPT), _f32),
            pltpu.VMEM((rw, NPT), _f32),
            pltpu.VMEM((NPT,), _f32),
            pltpu.VMEM((ncoef, 16), _f32),
            pltpu.VMEM((NPT,), _f32),
            pltpu.SemaphoreType.DMA,
        ],
    )


def kernel(x, edge_index, W1, att_src1, att_dst1, bias1,
           W2, att_src2, att_dst2, bias2):
    xf = x.reshape(N).astype(_f32)
    src = edge_index[0].astype(_i32).reshape(NW * NCH, K)
    dst = edge_index[1].astype(_i32).reshape(NW * NCH, K)

    W1r = W1.reshape(4, 64)
    W2r = W2.reshape(4, 64)
    cs1 = jnp.sum(W1r * att_src1[0], axis=1)
    cd1 = jnp.sum(W1r * att_dst1[0], axis=1)
    prod = W1r * W2r
    pco = jnp.sum(jnp.where(W1r > 0, prod, 0.0), axis=1)
    mco = jnp.sum(jnp.where(W1r < 0, prod, 0.0), axis=1)
    cs2 = att_src2.reshape(())
    cd2 = att_dst2.reshape(())

    ones16 = jnp.ones((16,), _f32)
    coef_e1 = jnp.concatenate([cs1, cd1])[:, None] * ones16
    coef_n1 = jnp.concatenate([cs1 + cd1, pco, mco])[:, None] * ones16
    coef_e2 = jnp.stack([cs2, cd2])[:, None] * ones16
    coef_n2 = jnp.stack([cs2 + cd2, bias2.reshape(())])[:, None] * ones16

    edge1 = _make_edge_pass(4)
    edge2 = _make_edge_pass(1)
    node1 = _make_node_pass(_node1_body, 12, 8)
    node2 = _make_node_pass(_node2_body, 2, 2)

    acc1 = edge1(xf, src, dst, coef_e1)
    xp2 = node1(acc1, xf, coef_n1)
    acc2 = edge2(xp2, src, dst, coef_e2)
    out = node2(acc2, xp2, coef_n2)
    return out.reshape(1, 256, 256)
